# baseline (speedup 1.0000x reference)
import functools

import jax
import jax.numpy as jnp
from jax.experimental import pallas as pl
from jax.experimental.pallas import tpu as pltpu

_NEG_SLOPE = 0.01
_NROWS = 2048
_NCOLS = 2048
_DN0 = (((0,), (0,)), ((), ()))
_DN1 = (((1,), (1,)), ((), ()))


def _fused_kernel(row_ref, col_ref, idx_ref, vals_ref, w_ref, b_ref, out_ref,
                  colsum_ref, rowsum_ref, wt_ref, beff_ref, *, nnz):
    p = pl.program_id(0)
    j = pl.program_id(1)
    tile, d = vals_ref.shape
    vals = vals_ref[...]

    @pl.when((p == 0) & (j == 0))
    def _init():
        colsum_ref[...] = jnp.zeros_like(colsum_ref)
        rowsum_ref[...] = jnp.zeros_like(rowsum_ref)

    @pl.when(p == 0)
    def _scatter():
        ci = jax.lax.broadcasted_iota(jnp.int32, (tile, _NCOLS), 1)
        oh_c = (col_ref[...] == ci).astype(jnp.float32)
        ri = jax.lax.broadcasted_iota(jnp.int32, (tile, _NROWS), 1)
        oh_r = (row_ref[...] == ri).astype(jnp.float32)
        colsum_ref[...] += jax.lax.dot_general(
            vals, oh_c, _DN0, preferred_element_type=jnp.float32)
        rowsum_ref[...] += jax.lax.dot_general(
            vals, oh_r, _DN0, preferred_element_type=jnp.float32)

    @pl.when((p == 1) & (j == 0))
    def _prepare():
        wt_ref[...] = w_ref[...].T
        w3t = wt_ref[:, 3 * d:4 * d]
        vsum_t = jnp.sum(colsum_ref[...], axis=1, keepdims=True)
        beff = jnp.dot(w3t, vsum_t / nnz,
                       preferred_element_type=jnp.float32)
        beff_ref[...] = jnp.broadcast_to(beff + b_ref[...].T,
                                         beff_ref.shape)

    @pl.when(p == 1)
    def _gather():
        row = idx_ref[0:1, :]
        col = idx_ref[1:2, :]
        ci = jax.lax.broadcasted_iota(jnp.int32, (_NCOLS, tile), 0)
        oh_ct = (col == ci).astype(jnp.float32)
        ri = jax.lax.broadcasted_iota(jnp.int32, (_NROWS, tile), 0)
        oh_rt = (row == ri).astype(jnp.float32)
        g_c = jnp.dot(colsum_ref[...], oh_ct,
                      preferred_element_type=jnp.float32)
        g_r = jnp.dot(rowsum_ref[...], oh_rt,
                      preferred_element_type=jnp.float32)
        out_t = jax.lax.dot_general(
            wt_ref[:, 0:d], vals, _DN1,
            preferred_element_type=jnp.float32)
        out_t = out_t + jnp.dot(wt_ref[:, d:2 * d], g_c,
                                preferred_element_type=jnp.float32)
        out_t = out_t + jnp.dot(wt_ref[:, 2 * d:3 * d], g_r,
                                preferred_element_type=jnp.float32)
        out_t = out_t + beff_ref[:, 0:1]
        out_t = jnp.where(out_t >= 0.0, out_t, _NEG_SLOPE * out_t)
        out_ref[...] = out_t.T


def _forward(indices, values, w_t, b):
    nnz, d = values.shape
    out_dim = w_t.shape[1]

    idx = indices.astype(jnp.int32)
    row_t = idx[0][:, None]
    col_t = idx[1][:, None]
    w = w_t.astype(jnp.float32)
    b2 = b.astype(jnp.float32)[None, :]

    tile = 2048
    while nnz % tile != 0:
        tile //= 2
    nt = nnz // tile

    out = pl.pallas_call(
        functools.partial(_fused_kernel, nnz=nnz),
        out_shape=jax.ShapeDtypeStruct((nnz, out_dim), jnp.float32),
        grid=(2, nt),
        in_specs=[pl.BlockSpec((tile, 1), lambda p, j: (j, 0)),
                  pl.BlockSpec((tile, 1), lambda p, j: (j, 0)),
                  pl.BlockSpec((2, tile), lambda p, j: (0, j)),
                  pl.BlockSpec((tile, d), lambda p, j: (j, 0)),
                  pl.BlockSpec((4 * d, out_dim), lambda p, j: (0, 0)),
                  pl.BlockSpec((1, out_dim), lambda p, j: (0, 0))],
        out_specs=pl.BlockSpec((tile, out_dim), lambda p, j: (p * j, 0)),
        scratch_shapes=[pltpu.VMEM((d, _NCOLS), jnp.float32),
                        pltpu.VMEM((d, _NROWS), jnp.float32),
                        pltpu.VMEM((out_dim, 4 * d), jnp.float32),
                        pltpu.VMEM((out_dim, 128), jnp.float32)],
        compiler_params=pltpu.CompilerParams(
            dimension_semantics=("arbitrary", "arbitrary")),
    )(row_t, col_t, idx, values, w, b2)
    return out


def kernel(indices, values, w_t, b):
    return _forward(indices, values, w_t, b)


# speedup vs baseline: 1.0676x; 1.0676x over previous
import functools

import jax
import jax.numpy as jnp
from jax.experimental import pallas as pl
from jax.experimental.pallas import tpu as pltpu

_NEG_SLOPE = 0.01
_NROWS = 2048
_NCOLS = 2048
_DN0 = (((0,), (0,)), ((), ()))
_DN1 = (((1,), (1,)), ((), ()))


def _scatter_kernel(row_ref, col_ref, vals_ref, colsum_ref, rowsum_ref):
    @pl.when(pl.program_id(0) == 0)
    def _init():
        colsum_ref[...] = jnp.zeros_like(colsum_ref)
        rowsum_ref[...] = jnp.zeros_like(rowsum_ref)

    vals = vals_ref[...]
    tile = vals.shape[0]
    ci = jax.lax.broadcasted_iota(jnp.int32, (tile, _NCOLS), 1)
    oh_c = (col_ref[...] == ci).astype(jnp.float32)
    ri = jax.lax.broadcasted_iota(jnp.int32, (tile, _NROWS), 1)
    oh_r = (row_ref[...] == ri).astype(jnp.float32)
    colsum_ref[...] += jax.lax.dot_general(
        vals, oh_c, _DN0, preferred_element_type=jnp.float32)
    rowsum_ref[...] += jax.lax.dot_general(
        vals, oh_r, _DN0, preferred_element_type=jnp.float32)


def _gather_kernel(idx_ref, vals_ref, colsum_ref, rowsum_ref,
                   w_ref, b_ref, out_ref, wt_ref, beff_ref, *, nnz):
    tile, d = vals_ref.shape

    @pl.when(pl.program_id(0) == 0)
    def _prepare():
        wt_ref[...] = w_ref[...].T
        w3t = wt_ref[:, 3 * d:4 * d]
        vsum_t = jnp.sum(colsum_ref[...], axis=1, keepdims=True)
        beff = jnp.dot(w3t, vsum_t / nnz,
                       preferred_element_type=jnp.float32)
        beff_ref[...] = jnp.broadcast_to(beff + b_ref[...].T,
                                         beff_ref.shape)

    vals = vals_ref[...]
    row = idx_ref[0:1, :]
    col = idx_ref[1:2, :]
    ci = jax.lax.broadcasted_iota(jnp.int32, (_NCOLS, tile), 0)
    oh_ct = (col == ci).astype(jnp.float32)
    ri = jax.lax.broadcasted_iota(jnp.int32, (_NROWS, tile), 0)
    oh_rt = (row == ri).astype(jnp.float32)
    g_c = jnp.dot(colsum_ref[...], oh_ct,
                  preferred_element_type=jnp.float32)
    g_r = jnp.dot(rowsum_ref[...], oh_rt,
                  preferred_element_type=jnp.float32)
    out_t = jax.lax.dot_general(
        wt_ref[:, 0:d], vals, _DN1,
        preferred_element_type=jnp.float32)
    out_t = out_t + jnp.dot(wt_ref[:, d:2 * d], g_c,
                            preferred_element_type=jnp.float32)
    out_t = out_t + jnp.dot(wt_ref[:, 2 * d:3 * d], g_r,
                            preferred_element_type=jnp.float32)
    out_t = out_t + beff_ref[:, 0:1]
    out_t = jnp.where(out_t >= 0.0, out_t, _NEG_SLOPE * out_t)
    out_ref[...] = out_t.T


def _forward(indices, values, w_t, b):
    nnz, d = values.shape
    out_dim = w_t.shape[1]

    idx = indices.astype(jnp.int32)
    row_t = idx[0][:, None]
    col_t = idx[1][:, None]
    w = w_t.astype(jnp.float32)
    b2 = b.astype(jnp.float32)[None, :]

    tile1 = 8192
    while nnz % tile1 != 0:
        tile1 //= 2
    nt1 = nnz // tile1

    colsum_t, rowsum_t = pl.pallas_call(
        _scatter_kernel,
        out_shape=(jax.ShapeDtypeStruct((d, _NCOLS), jnp.float32),
                   jax.ShapeDtypeStruct((d, _NROWS), jnp.float32)),
        grid=(nt1,),
        in_specs=[pl.BlockSpec((tile1, 1), lambda i: (i, 0)),
                  pl.BlockSpec((tile1, 1), lambda i: (i, 0)),
                  pl.BlockSpec((tile1, d), lambda i: (i, 0))],
        out_specs=(pl.BlockSpec((d, _NCOLS), lambda i: (0, 0)),
                   pl.BlockSpec((d, _NROWS), lambda i: (0, 0))),
        compiler_params=pltpu.CompilerParams(
            dimension_semantics=("arbitrary",)),
    )(row_t, col_t, values)

    tile2 = 8192
    while nnz % tile2 != 0:
        tile2 //= 2
    nt2 = nnz // tile2

    out = pl.pallas_call(
        functools.partial(_gather_kernel, nnz=nnz),
        out_shape=jax.ShapeDtypeStruct((nnz, out_dim), jnp.float32),
        grid=(nt2,),
        in_specs=[pl.BlockSpec((2, tile2), lambda i: (0, i)),
                  pl.BlockSpec((tile2, d), lambda i: (i, 0)),
                  pl.BlockSpec((d, _NCOLS), lambda i: (0, 0)),
                  pl.BlockSpec((d, _NROWS), lambda i: (0, 0)),
                  pl.BlockSpec((4 * d, out_dim), lambda i: (0, 0)),
                  pl.BlockSpec((1, out_dim), lambda i: (0, 0))],
        out_specs=pl.BlockSpec((tile2, out_dim), lambda i: (i, 0)),
        scratch_shapes=[pltpu.VMEM((out_dim, 4 * d), jnp.float32),
                        pltpu.VMEM((out_dim, 128), jnp.float32)],
        compiler_params=pltpu.CompilerParams(
            dimension_semantics=("arbitrary",)),
    )(idx, values, colsum_t, rowsum_t, w, b2)
    return out


def kernel(indices, values, w_t, b):
    return _forward(indices, values, w_t, b)
